# baseline (speedup 1.0000x reference)
import jax
import jax.numpy as jnp
from jax import lax
from jax.experimental import pallas as pl
from jax.experimental.pallas import tpu as pltpu
from jax.experimental.pallas import tpu_sc as plsc

B = 16384
D = 16
NC = 2
NS = 16
NW = NC * NS
BPW = B // NW
BLK = 16
NBLK = BPW // BLK
GSUP = 12500


def _sc_kernel(cell_idx_hbm, gene_idx_hbm, params_hbm, cell_t_hbm,
               gene_lin_hbm, out_hbm, ibuf, cbuf, gbuf, fbuf, sems):
    wid = lax.axis_index("s") * NC + lax.axis_index("c")
    base = wid * BPW

    pltpu.sync_copy(cell_idx_hbm.at[pl.ds(base, BPW)], ibuf.at[pl.ds(0, BPW)])
    pltpu.sync_copy(gene_idx_hbm.at[pl.ds(base, BPW)],
                    ibuf.at[pl.ds(BPW, BPW)])
    pltpu.sync_copy(params_hbm, fbuf.at[pl.ds(BPW, 2 * D)])

    lanes = lax.iota(jnp.int32, D)

    def issue(blk, slot):
        col = blk * BLK
        civ = ibuf[pl.ds(col, BLK)]
        giv = ibuf[pl.ds(BPW + col, BLK)]
        cst = (civ >> 7) << 7
        for j in range(BLK):
            start = pl.multiple_of(cst[j], 128)
            pltpu.make_async_copy(
                cell_t_hbm.at[:, pl.ds(start, 128)],
                cbuf.at[slot, j], sems.at[slot]).start()
        sup_ref = ibuf.at[pl.ds(2 * BPW + BLK * slot, BLK)]
        sup_ref[...] = giv >> 3
        pltpu.make_async_copy(
            gene_lin_hbm.at[sup_ref], gbuf.at[slot], sems.at[slot]).start()

    def drain(slot):
        for j in range(BLK):
            pltpu.make_async_copy(
                cell_t_hbm.at[:, pl.ds(0, 128)],
                cbuf.at[slot, j], sems.at[slot]).wait()
        sup_ref = ibuf.at[pl.ds(2 * BPW + BLK * slot, BLK)]
        pltpu.make_async_copy(
            gene_lin_hbm.at[sup_ref], gbuf.at[slot], sems.at[slot]).wait()

    w_vec = fbuf[pl.ds(BPW, D)]
    b_vec = fbuf[pl.ds(BPW + D, D)]

    def compute(blk, slot):
        col = blk * BLK
        civ = ibuf[pl.ds(col, BLK)]
        giv = ibuf[pl.ds(BPW + col, BLK)]
        ccol = civ & 127
        gcol = (giv & 7) << 4
        slot_v = jnp.full((BLK,), slot, jnp.int32)
        acc = jnp.zeros((BLK,), jnp.float32)
        for d in range(D):
            dvec = jnp.full((BLK,), d, jnp.int32)
            c = plsc.load_gather(cbuf, [slot_v, lanes, dvec, ccol])
            g = plsc.load_gather(gbuf, [slot_v, lanes, gcol + d])
            acc = acc + c * g
        z = acc * w_vec + b_vec
        fbuf[pl.ds(col, BLK)] = 1.0 / (1.0 + jnp.exp(-z))

    issue(0, 0)

    def body(blk, carry):
        slot = lax.rem(blk, 2)

        @pl.when(blk + 1 < NBLK)
        def _():
            pl.when(slot == 0)(lambda: issue(blk + 1, 1))
            pl.when(slot == 1)(lambda: issue(blk + 1, 0))

        pl.when(slot == 0)(lambda: drain(0))
        pl.when(slot == 1)(lambda: drain(1))
        pl.when(slot == 0)(lambda: compute(blk, 0))
        pl.when(slot == 1)(lambda: compute(blk, 1))
        return carry

    lax.fori_loop(0, NBLK, body, 0)

    pltpu.sync_copy(fbuf.at[pl.ds(0, BPW)], out_hbm.at[pl.ds(base, BPW)])


@jax.jit
def _run(cell_idx, gene_idx, params, cell_t, gene_lin):
    mesh = plsc.VectorSubcoreMesh(core_axis_name="c", subcore_axis_name="s")
    fn = pl.kernel(
        _sc_kernel,
        mesh=mesh,
        compiler_params=pltpu.CompilerParams(
            needs_layout_passes=False, use_tc_tiling_on_sc=True),
        out_type=jax.ShapeDtypeStruct((B,), jnp.float32),
        scratch_types=[
            pltpu.VMEM((2 * BPW + 2 * BLK,), jnp.int32),
            pltpu.VMEM((2, BLK, D, 128), jnp.float32),
            pltpu.VMEM((2, BLK, 128), jnp.float32),
            pltpu.VMEM((BPW + 2 * D,), jnp.float32),
            pltpu.SemaphoreType.DMA((2,)),
        ],
    )
    return fn(cell_idx, gene_idx, params, cell_t, gene_lin)


def kernel(cell_indices, gene_indices, cell_table, gene_table, dec_W, dec_b):
    params = jnp.concatenate([
        jnp.full((D,), dec_W[0, 0], jnp.float32),
        jnp.full((D,), dec_b[0], jnp.float32),
    ])
    gene_lin = gene_table.reshape(GSUP, 128)
    out = _run(cell_indices.astype(jnp.int32), gene_indices.astype(jnp.int32),
               params, cell_table.T, gene_lin)
    return out.reshape(B, 1)


# speedup vs baseline: 1.1318x; 1.1318x over previous
import jax
import jax.numpy as jnp
from jax import lax
from jax.experimental import pallas as pl
from jax.experimental.pallas import tpu as pltpu
from jax.experimental.pallas import tpu_sc as plsc

B = 16384
D = 16
NC = 2
NS = 16
NW = NC * NS
BPW = B // NW
BLK = 16
NBLK = BPW // BLK
GCH = 782
GSUP = GCH * 16
OFFB = 2 * BPW


def _sc_kernel(cell_idx_hbm, gene_idx_hbm, params_hbm, cell_t_hbm,
               gene_t_hbm, out_hbm, gsp, ibuf, cbuf, gval, fbuf, ta, tb,
               sems, fsem, osem):
    wid = lax.axis_index("s") * NC + lax.axis_index("c")
    sid = lax.axis_index("s")
    base = wid * BPW

    pltpu.sync_copy(cell_idx_hbm.at[pl.ds(base, BPW)], ibuf.at[pl.ds(0, BPW)])
    pltpu.sync_copy(gene_idx_hbm.at[pl.ds(base, BPW)],
                    ibuf.at[pl.ds(BPW, BPW)])
    pltpu.sync_copy(params_hbm, fbuf.at[pl.ds(BPW, 2 * D)])

    lanes = lax.iota(jnp.int32, D)

    def issue_cell(blk, slot):
        col = blk * BLK
        civ = ibuf[pl.ds(col, BLK)]
        cst = (civ >> 7) << 7
        for j in range(BLK):
            start = pl.multiple_of(cst[j], 128)
            pltpu.make_async_copy(
                cell_t_hbm.at[:, pl.ds(start, 128)],
                cbuf.at[slot, j], sems.at[slot]).start()

    issue_cell(0, 0)

    nfill = (GCH + NS - 1) // NS

    def chunk_in(i, p):
        c = sid + i * NS

        @pl.when(c < GCH)
        def _():
            start = pl.multiple_of(c * 128, 128)
            pltpu.make_async_copy(
                gene_t_hbm.at[:, pl.ds(start, 128)], ta.at[p], fsem).start()

    def fill(i, carry):
        p = lax.rem(i, 2)

        @pl.when(i + 1 < nfill)
        def _():
            pl.when(p == 0)(lambda: chunk_in(i + 1, 1))
            pl.when(p == 1)(lambda: chunk_in(i + 1, 0))

        c = sid + i * NS

        @pl.when(c < GCH)
        def _():
            def work(p):
                pltpu.make_async_copy(
                    gene_t_hbm.at[:, pl.ds(0, 128)], ta.at[p], fsem).wait()
                for cg in range(8):
                    gloc = cg * BLK + lanes
                    rowv = gloc >> 3
                    cbase = (gloc & 7) << 4
                    for d in range(D):
                        v = ta[p, d, pl.ds(cg * BLK, BLK)]
                        plsc.store_scatter(tb.at[p], [rowv, cbase + d], v)
                start = pl.multiple_of(c * BLK, 8)
                pltpu.make_async_copy(
                    tb.at[p], gsp.at[pl.ds(start, BLK), :], osem).start()

            pl.when(p == 0)(lambda: work(0))
            pl.when(p == 1)(lambda: work(1))
        return carry

    chunk_in(0, 0)
    lax.fori_loop(0, nfill, fill, 0)

    def out_drain(i, carry):
        c = sid + i * NS

        @pl.when(c < GCH)
        def _():
            pltpu.make_async_copy(
                tb.at[0], gsp.at[pl.ds(0, BLK), :], osem).wait()
        return carry

    lax.fori_loop(0, nfill, out_drain, 0)
    plsc.subcore_barrier()

    def issue_gene(blk, slot):
        col = blk * BLK
        sup = ibuf.at[pl.ds(OFFB + BLK * slot, BLK)]
        sup[...] = ibuf[pl.ds(BPW + col, BLK)] >> 3
        pltpu.make_async_copy(gsp.at[sup], gval.at[slot],
                              sems.at[slot]).start()

    def drain(slot):
        for j in range(BLK):
            pltpu.make_async_copy(
                cell_t_hbm.at[:, pl.ds(0, 128)],
                cbuf.at[slot, j], sems.at[slot]).wait()
        sup = ibuf.at[pl.ds(OFFB + BLK * slot, BLK)]
        pltpu.make_async_copy(gsp.at[sup], gval.at[slot],
                              sems.at[slot]).wait()

    w_vec = fbuf[pl.ds(BPW, D)]
    b_vec = fbuf[pl.ds(BPW + D, D)]

    def compute(blk, slot):
        col = blk * BLK
        civ = ibuf[pl.ds(col, BLK)]
        giv = ibuf[pl.ds(BPW + col, BLK)]
        ccol = civ & 127
        gcol = (giv & 7) << 4
        slot_v = jnp.full((BLK,), slot, jnp.int32)
        acc = jnp.zeros((BLK,), jnp.float32)
        for d in range(D):
            dvec = jnp.full((BLK,), d, jnp.int32)
            c = plsc.load_gather(cbuf, [slot_v, lanes, dvec, ccol])
            g = plsc.load_gather(gval, [slot_v, lanes, gcol + d])
            acc = acc + c * g
        z = acc * w_vec + b_vec
        fbuf[pl.ds(col, BLK)] = 1.0 / (1.0 + jnp.exp(-z))

    issue_gene(0, 0)

    def body(blk, carry):
        slot = lax.rem(blk, 2)

        @pl.when(blk + 1 < NBLK)
        def _():
            pl.when(slot == 0)(lambda: issue_cell(blk + 1, 1))
            pl.when(slot == 1)(lambda: issue_cell(blk + 1, 0))
            pl.when(slot == 0)(lambda: issue_gene(blk + 1, 1))
            pl.when(slot == 1)(lambda: issue_gene(blk + 1, 0))

        pl.when(slot == 0)(lambda: drain(0))
        pl.when(slot == 1)(lambda: drain(1))
        pl.when(slot == 0)(lambda: compute(blk, 0))
        pl.when(slot == 1)(lambda: compute(blk, 1))
        return carry

    lax.fori_loop(0, NBLK, body, 0)

    pltpu.sync_copy(fbuf.at[pl.ds(0, BPW)], out_hbm.at[pl.ds(base, BPW)])


@jax.jit
def _run(cell_idx, gene_idx, params, cell_t, gene_t):
    mesh = plsc.VectorSubcoreMesh(core_axis_name="c", subcore_axis_name="s")
    fn = pl.kernel(
        _sc_kernel,
        mesh=mesh,
        compiler_params=pltpu.CompilerParams(
            needs_layout_passes=False, use_tc_tiling_on_sc=True),
        out_type=(jax.ShapeDtypeStruct((B,), jnp.float32),
                  jax.ShapeDtypeStruct((GSUP, 128), jnp.float32)),
        scratch_types=[
            pltpu.VMEM((OFFB + 2 * BLK,), jnp.int32),
            pltpu.VMEM((2, BLK, D, 128), jnp.float32),
            pltpu.VMEM((2, BLK, 128), jnp.float32),
            pltpu.VMEM((BPW + 2 * D,), jnp.float32),
            pltpu.VMEM((2, D, 128), jnp.float32),
            pltpu.VMEM((2, D, 128), jnp.float32),
            pltpu.SemaphoreType.DMA((2,)),
            pltpu.SemaphoreType.DMA,
            pltpu.SemaphoreType.DMA,
        ],
    )
    return fn(cell_idx, gene_idx, params, cell_t, gene_t)[0]


def kernel(cell_indices, gene_indices, cell_table, gene_table, dec_W, dec_b):
    params = jnp.concatenate([
        jnp.full((D,), dec_W[0, 0], jnp.float32),
        jnp.full((D,), dec_b[0], jnp.float32),
    ])
    out = _run(cell_indices.astype(jnp.int32), gene_indices.astype(jnp.int32),
               params, cell_table.T, gene_table.T)
    return out.reshape(B, 1)


# speedup vs baseline: 1.1758x; 1.0389x over previous
import jax
import jax.numpy as jnp
from jax import lax
from jax.experimental import pallas as pl
from jax.experimental.pallas import tpu as pltpu
from jax.experimental.pallas import tpu_sc as plsc

B = 16384
D = 16
NC = 2
NS = 16
NW = NC * NS
BPW = B // NW
BLK = 16
NBLK = BPW // BLK
GCH = 782
GSUP = GCH * 16
OFFB = 2 * BPW


def _sc_kernel(cell_idx_hbm, gene_idx_hbm, params_hbm, cell_t_hbm,
               gene_t_hbm, out_hbm, gsp, ibuf, cbuf, gval, fbuf, ta, tb,
               sems, fsem, osem):
    wid = lax.axis_index("s") * NC + lax.axis_index("c")
    sid = lax.axis_index("s")
    base = wid * BPW

    pltpu.sync_copy(cell_idx_hbm.at[pl.ds(base, BPW)], ibuf.at[pl.ds(0, BPW)])
    pltpu.sync_copy(gene_idx_hbm.at[pl.ds(base, BPW)],
                    ibuf.at[pl.ds(BPW, BPW)])
    pltpu.sync_copy(params_hbm, fbuf.at[pl.ds(BPW, 2 * D)])

    lanes = lax.iota(jnp.int32, D)

    def issue_cell(blk, slot):
        col = blk * BLK
        civ = ibuf[pl.ds(col, BLK)]
        cst = (civ >> 7) << 7
        for j in range(BLK):
            start = pl.multiple_of(cst[j], 128)
            pltpu.make_async_copy(
                cell_t_hbm.at[:, pl.ds(start, 128)],
                cbuf.at[slot, j], sems.at[slot]).start()

    issue_cell(0, 0)
    issue_cell(1, 1)

    nfill = (GCH + NS - 1) // NS

    def chunk_in(i, p):
        c = sid + i * NS

        @pl.when(c < GCH)
        def _():
            start = pl.multiple_of(c * 128, 128)
            pltpu.make_async_copy(
                gene_t_hbm.at[:, pl.ds(start, 128)], ta.at[p], fsem).start()

    def fill(i, carry):
        p = lax.rem(i, 2)

        @pl.when(i + 1 < nfill)
        def _():
            pl.when(p == 0)(lambda: chunk_in(i + 1, 1))
            pl.when(p == 1)(lambda: chunk_in(i + 1, 0))

        c = sid + i * NS

        @pl.when(c < GCH)
        def _():
            def work(p):
                pltpu.make_async_copy(
                    gene_t_hbm.at[:, pl.ds(0, 128)], ta.at[p], fsem).wait()
                for cg in range(8):
                    gloc = cg * BLK + lanes
                    rowv = gloc >> 3
                    cbase = (gloc & 7) << 4
                    for d in range(D):
                        v = ta[p, d, pl.ds(cg * BLK, BLK)]
                        plsc.store_scatter(tb.at[p], [rowv, cbase + d], v)
                start = pl.multiple_of(c * BLK, 8)
                pltpu.make_async_copy(
                    tb.at[p], gsp.at[pl.ds(start, BLK), :], osem).start()

            pl.when(p == 0)(lambda: work(0))
            pl.when(p == 1)(lambda: work(1))
        return carry

    chunk_in(0, 0)
    lax.fori_loop(0, nfill, fill, 0)

    def out_drain(i, carry):
        c = sid + i * NS

        @pl.when(c < GCH)
        def _():
            pltpu.make_async_copy(
                tb.at[0], gsp.at[pl.ds(0, BLK), :], osem).wait()
        return carry

    lax.fori_loop(0, nfill, out_drain, 0)
    plsc.subcore_barrier()

    def issue_gene(blk, slot):
        col = blk * BLK
        sup = ibuf.at[pl.ds(OFFB + BLK * slot, BLK)]
        sup[...] = ibuf[pl.ds(BPW + col, BLK)] >> 3
        pltpu.make_async_copy(gsp.at[sup], gval.at[slot],
                              sems.at[slot]).start()

    def drain(slot):
        for j in range(BLK):
            pltpu.make_async_copy(
                cell_t_hbm.at[:, pl.ds(0, 128)],
                cbuf.at[slot, j], sems.at[slot]).wait()
        sup = ibuf.at[pl.ds(OFFB + BLK * slot, BLK)]
        pltpu.make_async_copy(gsp.at[sup], gval.at[slot],
                              sems.at[slot]).wait()

    w_vec = fbuf[pl.ds(BPW, D)]
    b_vec = fbuf[pl.ds(BPW + D, D)]

    def compute(blk, slot):
        col = blk * BLK
        civ = ibuf[pl.ds(col, BLK)]
        giv = ibuf[pl.ds(BPW + col, BLK)]
        ccol = civ & 127
        gcol = (giv & 7) << 4
        slot_v = jnp.full((BLK,), slot, jnp.int32)
        acc = jnp.zeros((BLK,), jnp.float32)
        for d in range(D):
            dvec = jnp.full((BLK,), d, jnp.int32)
            c = plsc.load_gather(cbuf, [slot_v, lanes, dvec, ccol])
            g = plsc.load_gather(gval, [slot_v, lanes, gcol + d])
            acc = acc + c * g
        z = acc * w_vec + b_vec
        fbuf[pl.ds(col, BLK)] = 1.0 / (1.0 + jnp.exp(-z))

    issue_gene(0, 0)
    issue_gene(1, 1)

    def body(blk, carry):
        slot = lax.rem(blk, 3)

        @pl.when(blk + 2 < NBLK)
        def _():
            pl.when(slot == 0)(lambda: issue_cell(blk + 2, 2))
            pl.when(slot == 1)(lambda: issue_cell(blk + 2, 0))
            pl.when(slot == 2)(lambda: issue_cell(blk + 2, 1))
            pl.when(slot == 0)(lambda: issue_gene(blk + 2, 2))
            pl.when(slot == 1)(lambda: issue_gene(blk + 2, 0))
            pl.when(slot == 2)(lambda: issue_gene(blk + 2, 1))

        pl.when(slot == 0)(lambda: drain(0))
        pl.when(slot == 1)(lambda: drain(1))
        pl.when(slot == 2)(lambda: drain(2))
        pl.when(slot == 0)(lambda: compute(blk, 0))
        pl.when(slot == 1)(lambda: compute(blk, 1))
        pl.when(slot == 2)(lambda: compute(blk, 2))
        return carry

    lax.fori_loop(0, NBLK, body, 0)

    pltpu.sync_copy(fbuf.at[pl.ds(0, BPW)], out_hbm.at[pl.ds(base, BPW)])


@jax.jit
def _run(cell_idx, gene_idx, params, cell_t, gene_t):
    mesh = plsc.VectorSubcoreMesh(core_axis_name="c", subcore_axis_name="s")
    fn = pl.kernel(
        _sc_kernel,
        mesh=mesh,
        compiler_params=pltpu.CompilerParams(
            needs_layout_passes=False, use_tc_tiling_on_sc=True),
        out_type=(jax.ShapeDtypeStruct((B,), jnp.float32),
                  jax.ShapeDtypeStruct((GSUP, 128), jnp.float32)),
        scratch_types=[
            pltpu.VMEM((OFFB + 3 * BLK,), jnp.int32),
            pltpu.VMEM((3, BLK, D, 128), jnp.float32),
            pltpu.VMEM((3, BLK, 128), jnp.float32),
            pltpu.VMEM((BPW + 2 * D,), jnp.float32),
            pltpu.VMEM((2, D, 128), jnp.float32),
            pltpu.VMEM((2, D, 128), jnp.float32),
            pltpu.SemaphoreType.DMA((3,)),
            pltpu.SemaphoreType.DMA,
            pltpu.SemaphoreType.DMA,
        ],
    )
    return fn(cell_idx, gene_idx, params, cell_t, gene_t)[0]


def kernel(cell_indices, gene_indices, cell_table, gene_table, dec_W, dec_b):
    params = jnp.concatenate([
        jnp.full((D,), dec_W[0, 0], jnp.float32),
        jnp.full((D,), dec_b[0], jnp.float32),
    ])
    out = _run(cell_indices.astype(jnp.int32), gene_indices.astype(jnp.int32),
               params, cell_table.T, gene_table.T)
    return out.reshape(B, 1)
